# baseline (speedup 1.0000x reference)
import jax
import jax.numpy as jnp
from jax import lax
from jax.experimental import pallas as pl
from jax.experimental.pallas import tpu as pltpu
from jax.experimental.pallas import tpu_sc as plsc

N_NODES = 10000
N_EDGES = 320000
IN_DIM = 128
HID = 64
NUM_CLASSES = 2
NUM_GRAPHS = 512

_NC = 2
_NS = 16
_NW = _NC * _NS
_CHUNK = 96
_NCHUNK = 105
_PER_W = _NCHUNK * _CHUNK
_E_PAD = _NW * _PER_W
_N_PAD = 10112
_ROWS_PER_TILE = _N_PAD // _NS


def _mid_body(p_ref, x_ref, wr1_ref, b1_ref, wo1_ref, wr2p_ref, wo2_ref,
              b2_ref, xr2p_ref, xo2_ref):
    agg1 = p_ref[0, :N_NODES] + p_ref[1, :N_NODES]
    h1 = jnp.maximum(
        jnp.dot(agg1, wr1_ref[...], preferred_element_type=jnp.float32)
        + b1_ref[...]
        + jnp.dot(x_ref[...], wo1_ref[...], preferred_element_type=jnp.float32),
        0.0)
    xr2p_ref[...] = jnp.dot(h1, wr2p_ref[...],
                            preferred_element_type=jnp.float32)
    xo2_ref[...] = (jnp.dot(h1, wo2_ref[...],
                            preferred_element_type=jnp.float32) + b2_ref[...])


def _pool_body(p_ref, xo2_ref, batch_ref, wl_ref, bl_ref, out_ref):
    h2 = jnp.maximum(p_ref[0, :N_NODES, :HID] + p_ref[1, :N_NODES, :HID]
                     + xo2_ref[...], 0.0)
    gid = lax.broadcasted_iota(jnp.int32, (NUM_GRAPHS, 1), 0)
    sums = jnp.zeros((NUM_GRAPHS, HID), jnp.float32)
    counts = jnp.zeros((NUM_GRAPHS, 1), jnp.float32)
    ch = 1000
    for k in range(N_NODES // ch):
        b = batch_ref[:, k * ch:(k + 1) * ch]
        m = (gid == b).astype(jnp.float32)
        sums = sums + jnp.dot(m, h2[k * ch:(k + 1) * ch, :],
                              preferred_element_type=jnp.float32,
                              precision=lax.Precision.HIGHEST)
        counts = counts + jnp.sum(m, axis=1, keepdims=True)
    pooled = sums / jnp.maximum(counts, 1.0)
    out_ref[...] = (jnp.dot(pooled, wl_ref[...],
                            preferred_element_type=jnp.float32,
                            precision=lax.Precision.HIGHEST) + bl_ref[...])


_mid = pl.pallas_call(
    _mid_body,
    out_shape=[jax.ShapeDtypeStruct((N_NODES, IN_DIM), jnp.float32),
               jax.ShapeDtypeStruct((N_NODES, HID), jnp.float32)],
)

_pool = pl.pallas_call(
    _pool_body,
    out_shape=jax.ShapeDtypeStruct((NUM_GRAPHS, NUM_CLASSES), jnp.float32),
)


def _sc_scatter_body(xr_hbm, src_hbm, dst_hbm, zeros_hbm, out_hbm,
                     src_all, dst_all, b0, b1, acc_sh, g0, g1):
    cid = lax.axis_index("c")
    sid = lax.axis_index("s")
    wid = cid * _NS + sid
    row0 = sid * _ROWS_PER_TILE

    pltpu.sync_copy(src_hbm.at[wid], src_all)
    pltpu.sync_copy(dst_hbm.at[wid], dst_all)
    pltpu.sync_copy(zeros_hbm.at[pl.ds(row0, _ROWS_PER_TILE)],
                    acc_sh.at[pl.ds(row0, _ROWS_PER_TILE)])
    plsc.subcore_barrier()

    def gather(j, buf, sem):
        pltpu.async_copy(
            xr_hbm.at[src_all.at[pl.ds(pl.multiple_of(j * _CHUNK, 8),
                                       _CHUNK)]], buf, sem)

    def wait_gather(buf, sem):
        pltpu.make_async_copy(xr_hbm.at[src_all.at[pl.ds(0, _CHUNK)]],
                              buf, sem).wait()

    def scatter(j, buf):
        pltpu.sync_copy(buf, acc_sh.at[dst_all.at[j]], add=True)

    gather(0, b0, g0)

    def body(jo, carry):
        j = 2 * jo
        gather(j + 1, b1, g1)
        wait_gather(b0, g0)
        scatter(j, b0)
        gather(j + 2, b0, g0)
        wait_gather(b1, g1)
        scatter(j + 1, b1)
        return carry

    lax.fori_loop(0, (_NCHUNK - 1) // 2, body, 0)

    wait_gather(b0, g0)
    scatter(_NCHUNK - 1, b0)

    plsc.subcore_barrier()
    pltpu.sync_copy(acc_sh.at[pl.ds(row0, _ROWS_PER_TILE)],
                    out_hbm.at[cid, pl.ds(row0, _ROWS_PER_TILE)])


_SC_SCATTER_CACHE = []


def _get_sc_scatter():
    if not _SC_SCATTER_CACHE:
        _SC_SCATTER_CACHE.append(pl.kernel(
            _sc_scatter_body,
            out_type=jax.ShapeDtypeStruct((_NC, _N_PAD, IN_DIM),
                                          jnp.float32),
            mesh=plsc.VectorSubcoreMesh(core_axis_name="c",
                                        subcore_axis_name="s"),
            scratch_types=[
                pltpu.VMEM((_PER_W,), jnp.int32),
                pltpu.VMEM((_NCHUNK, _CHUNK), jnp.int32),
                pltpu.VMEM((_CHUNK, IN_DIM), jnp.float32),
                pltpu.VMEM((_CHUNK, IN_DIM), jnp.float32),
                pltpu.VMEM_SHARED((_N_PAD, IN_DIM), jnp.float32),
                pltpu.SemaphoreType.DMA,
                pltpu.SemaphoreType.DMA,
            ],
        ))
    return _SC_SCATTER_CACHE[0]


def kernel(x, edge_index, batch, W_rel1, b_rel1, W_root1,
           W_rel2, b_rel2, W_root2, W_lin, b_lin):
    npad = _E_PAD - N_EDGES
    src = jnp.concatenate(
        [edge_index[0].astype(jnp.int32), jnp.zeros((npad,), jnp.int32)]
    ).reshape(_NW, _PER_W)
    dst = jnp.concatenate(
        [edge_index[1].astype(jnp.int32),
         jnp.full((npad,), _N_PAD - 1, jnp.int32)]
    ).reshape(_NW, _NCHUNK, _CHUNK)
    batch2d = batch.astype(jnp.int32).reshape(1, N_NODES)
    zeros_pad = jnp.zeros((_N_PAD, IN_DIM), jnp.float32)
    W_rel2p = jnp.pad(W_rel2, ((0, 0), (0, IN_DIM - HID)))

    sc_scatter = _get_sc_scatter()
    p1 = sc_scatter(x, src, dst, zeros_pad)
    xr2p, xo2 = _mid(p1, x, W_rel1, b_rel1.reshape(1, HID), W_root1,
                     W_rel2p, W_root2, b_rel2.reshape(1, HID))
    p2 = sc_scatter(xr2p, src, dst, zeros_pad)
    return _pool(p2, xo2, batch2d, W_lin, b_lin.reshape(1, NUM_CLASSES))


# speedup vs baseline: 1.0011x; 1.0011x over previous
import jax
import jax.numpy as jnp
from jax import lax
from jax.experimental import pallas as pl
from jax.experimental.pallas import tpu as pltpu
from jax.experimental.pallas import tpu_sc as plsc

N_NODES = 10000
N_EDGES = 320000
IN_DIM = 128
HID = 64
NUM_CLASSES = 2
NUM_GRAPHS = 512

_NC = 2
_NS = 16
_NW = _NC * _NS
_CHUNK = 96
_NCHUNK = 105
_PER_W = _NCHUNK * _CHUNK
_E_PAD = _NW * _PER_W
_N_PAD = 10112
_ROWS_PER_TILE = _N_PAD // _NS


def _mid_body(p_ref, x_ref, wr1_ref, b1_ref, wo1_ref, wr2p_ref, wo2_ref,
              b2_ref, xr2p_ref, xo2_ref):
    agg1 = p_ref[0, :N_NODES] + p_ref[1, :N_NODES]
    h1 = jnp.maximum(
        jnp.dot(agg1, wr1_ref[...], preferred_element_type=jnp.float32)
        + b1_ref[...]
        + jnp.dot(x_ref[...], wo1_ref[...], preferred_element_type=jnp.float32),
        0.0)
    xr2p_ref[...] = jnp.dot(h1, wr2p_ref[...],
                            preferred_element_type=jnp.float32)
    xo2_ref[...] = (jnp.dot(h1, wo2_ref[...],
                            preferred_element_type=jnp.float32) + b2_ref[...])


def _pool_body(p_ref, xo2_ref, batch_ref, wl_ref, bl_ref, out_ref):
    h2 = jnp.maximum(p_ref[0, :N_NODES, :HID] + p_ref[1, :N_NODES, :HID]
                     + xo2_ref[...], 0.0)
    gid = lax.broadcasted_iota(jnp.int32, (NUM_GRAPHS, 1), 0)
    sums = jnp.zeros((NUM_GRAPHS, HID), jnp.float32)
    counts = jnp.zeros((NUM_GRAPHS, 1), jnp.float32)
    ch = 1000
    for k in range(N_NODES // ch):
        b = batch_ref[:, k * ch:(k + 1) * ch]
        m = (gid == b).astype(jnp.float32)
        sums = sums + jnp.dot(m, h2[k * ch:(k + 1) * ch, :],
                              preferred_element_type=jnp.float32,
                              precision=lax.Precision.HIGHEST)
        counts = counts + jnp.sum(m, axis=1, keepdims=True)
    pooled = sums / jnp.maximum(counts, 1.0)
    out_ref[...] = (jnp.dot(pooled, wl_ref[...],
                            preferred_element_type=jnp.float32,
                            precision=lax.Precision.HIGHEST) + bl_ref[...])


_mid = pl.pallas_call(
    _mid_body,
    out_shape=[jax.ShapeDtypeStruct((N_NODES, IN_DIM), jnp.float32),
               jax.ShapeDtypeStruct((N_NODES, HID), jnp.float32)],
)

_pool = pl.pallas_call(
    _pool_body,
    out_shape=jax.ShapeDtypeStruct((NUM_GRAPHS, NUM_CLASSES), jnp.float32),
)


def _sc_scatter_body(xr_hbm, src_hbm, dst_hbm, zeros_hbm, out_hbm,
                     src_all, dst_all, b0, b1, acc_sh, g0, g1):
    cid = lax.axis_index("c")
    sid = lax.axis_index("s")
    wid = cid * _NS + sid
    row0 = sid * _ROWS_PER_TILE

    pltpu.sync_copy(src_hbm.at[wid], src_all)
    pltpu.sync_copy(dst_hbm.at[wid], dst_all)
    pltpu.sync_copy(zeros_hbm.at[pl.ds(row0, _ROWS_PER_TILE)],
                    acc_sh.at[pl.ds(row0, _ROWS_PER_TILE)])
    plsc.subcore_barrier()

    def gather(j, buf, sem):
        pltpu.async_copy(
            xr_hbm.at[src_all.at[pl.ds(pl.multiple_of(j * _CHUNK, 8),
                                       _CHUNK)]], buf, sem)

    def wait_gather(buf, sem):
        pltpu.make_async_copy(xr_hbm.at[src_all.at[pl.ds(0, _CHUNK)]],
                              buf, sem).wait()

    def scatter(j, buf):
        pltpu.sync_copy(buf, acc_sh.at[dst_all.at[j]], add=True)

    gather(0, b0, g0)

    def body(jo, carry):
        j = 2 * jo
        gather(j + 1, b1, g1)
        wait_gather(b0, g0)
        scatter(j, b0)
        gather(j + 2, b0, g0)
        wait_gather(b1, g1)
        scatter(j + 1, b1)
        return carry

    lax.fori_loop(0, (_NCHUNK - 1) // 2, body, 0)

    wait_gather(b0, g0)
    scatter(_NCHUNK - 1, b0)

    plsc.subcore_barrier()
    pltpu.sync_copy(acc_sh.at[pl.ds(row0, _ROWS_PER_TILE)],
                    out_hbm.at[cid, pl.ds(row0, _ROWS_PER_TILE)])


_SC_SCATTER_CACHE = []


def _get_sc_scatter():
    if not _SC_SCATTER_CACHE:
        _SC_SCATTER_CACHE.append(pl.kernel(
            _sc_scatter_body,
            out_type=jax.ShapeDtypeStruct((_NC, _N_PAD, IN_DIM),
                                          jnp.float32),
            mesh=plsc.VectorSubcoreMesh(core_axis_name="c",
                                        subcore_axis_name="s"),
            scratch_types=[
                pltpu.VMEM((_PER_W,), jnp.int32),
                pltpu.VMEM((_NCHUNK, _CHUNK), jnp.int32),
                pltpu.VMEM((_CHUNK, IN_DIM), jnp.float32),
                pltpu.VMEM((_CHUNK, IN_DIM), jnp.float32),
                pltpu.VMEM_SHARED((_N_PAD, IN_DIM), jnp.float32),
                pltpu.SemaphoreType.DMA,
                pltpu.SemaphoreType.DMA,
            ],
        ))
    return _SC_SCATTER_CACHE[0]


def kernel(x, edge_index, batch, W_rel1, b_rel1, W_root1,
           W_rel2, b_rel2, W_root2, W_lin, b_lin):
    npad = _E_PAD - N_EDGES
    src = jnp.concatenate(
        [edge_index[0].astype(jnp.int32), jnp.zeros((npad,), jnp.int32)]
    ).reshape(_NW, _PER_W)
    trash = N_NODES + jnp.arange(npad, dtype=jnp.int32) % (_N_PAD - N_NODES)
    dst = jnp.concatenate(
        [edge_index[1].astype(jnp.int32), trash]
    ).reshape(_NW, _NCHUNK, _CHUNK)
    batch2d = batch.astype(jnp.int32).reshape(1, N_NODES)
    zeros_pad = jnp.zeros((_N_PAD, IN_DIM), jnp.float32)
    W_rel2p = jnp.pad(W_rel2, ((0, 0), (0, IN_DIM - HID)))

    sc_scatter = _get_sc_scatter()
    p1 = sc_scatter(x, src, dst, zeros_pad)
    xr2p, xo2 = _mid(p1, x, W_rel1, b_rel1.reshape(1, HID), W_root1,
                     W_rel2p, W_root2, b_rel2.reshape(1, HID))
    p2 = sc_scatter(xr2p, src, dst, zeros_pad)
    return _pool(p2, xo2, batch2d, W_lin, b_lin.reshape(1, NUM_CLASSES))


# speedup vs baseline: 1.6306x; 1.6288x over previous
import jax
import jax.numpy as jnp
from jax import lax
from jax.experimental import pallas as pl
from jax.experimental.pallas import tpu as pltpu
from jax.experimental.pallas import tpu_sc as plsc

N_NODES = 10000
N_EDGES = 320000
IN_DIM = 128
HID = 64
NUM_CLASSES = 2
NUM_GRAPHS = 512

_NC = 2
_NS = 16
_NW = _NC * _NS
_PER_W = N_EDGES // _NW
_CHUNK = 80
_NCHUNK = _PER_W // _CHUNK
_N_PAD = 10240
_ROWS_PER_TILE = _N_PAD // _NS


def _pre_body(x_ref, wo1_ref, b1_ref, xo1_ref):
    xo1_ref[...] = (jnp.dot(x_ref[...], wo1_ref[...],
                            preferred_element_type=jnp.float32) + b1_ref[...])


def _mid_body(p_ref, xo1_ref, wr1_ref, wr2p_ref, wo2_ref,
              b2_ref, xr2p_ref, xo2_ref):
    agg1 = p_ref[0, :N_NODES] + p_ref[1, :N_NODES]
    h1 = jnp.maximum(
        jnp.dot(agg1, wr1_ref[...], preferred_element_type=jnp.float32)
        + xo1_ref[...],
        0.0)
    xr2p_ref[...] = jnp.dot(h1, wr2p_ref[...],
                            preferred_element_type=jnp.float32)
    xo2_ref[...] = (jnp.dot(h1, wo2_ref[...],
                            preferred_element_type=jnp.float32) + b2_ref[...])


def _pool_body(p_ref, xo2_ref, batch_ref, wl_ref, bl_ref, out_ref):
    h2 = jnp.maximum(p_ref[0, :N_NODES, :HID] + p_ref[1, :N_NODES, :HID]
                     + xo2_ref[...], 0.0)
    gid = lax.broadcasted_iota(jnp.int32, (NUM_GRAPHS, 1), 0)
    sums = jnp.zeros((NUM_GRAPHS, HID), jnp.float32)
    counts = jnp.zeros((NUM_GRAPHS, 1), jnp.float32)
    ch = 1000
    for k in range(N_NODES // ch):
        b = batch_ref[:, k * ch:(k + 1) * ch]
        m = (gid == b).astype(jnp.float32)
        sums = sums + jnp.dot(m, h2[k * ch:(k + 1) * ch, :],
                              preferred_element_type=jnp.float32,
                              precision=lax.Precision.HIGHEST)
        counts = counts + jnp.sum(m, axis=1, keepdims=True)
    pooled = sums / jnp.maximum(counts, 1.0)
    out_ref[...] = (jnp.dot(pooled, wl_ref[...],
                            preferred_element_type=jnp.float32,
                            precision=lax.Precision.HIGHEST) + bl_ref[...])


_pre = pl.pallas_call(
    _pre_body,
    out_shape=jax.ShapeDtypeStruct((N_NODES, HID), jnp.float32),
)

_mid = pl.pallas_call(
    _mid_body,
    out_shape=[jax.ShapeDtypeStruct((N_NODES, IN_DIM), jnp.float32),
               jax.ShapeDtypeStruct((N_NODES, HID), jnp.float32)],
)

_pool = pl.pallas_call(
    _pool_body,
    out_shape=jax.ShapeDtypeStruct((NUM_GRAPHS, NUM_CLASSES), jnp.float32),
)


def _sc_scatter_body(xr_hbm, src_hbm, dst_hbm, zeros_hbm, out_hbm,
                     src_all, dst_all, b0, b1, acc_sh, g0, g1):
    cid = lax.axis_index("c")
    sid = lax.axis_index("s")
    wid = cid * _NS + sid
    row0 = sid * _ROWS_PER_TILE

    pltpu.sync_copy(src_hbm.at[wid], src_all)
    pltpu.sync_copy(dst_hbm.at[wid], dst_all)
    pltpu.sync_copy(zeros_hbm.at[pl.ds(row0, _ROWS_PER_TILE)],
                    acc_sh.at[pl.ds(row0, _ROWS_PER_TILE)])
    plsc.subcore_barrier()

    def gather(j, buf, sem):
        pltpu.async_copy(
            xr_hbm.at[src_all.at[pl.ds(pl.multiple_of(j * _CHUNK, 8),
                                       _CHUNK)]], buf, sem)

    def wait_gather(buf, sem):
        pltpu.make_async_copy(xr_hbm.at[src_all.at[pl.ds(0, _CHUNK)]],
                              buf, sem).wait()

    def scatter(j, buf):
        pltpu.sync_copy(buf, acc_sh.at[dst_all.at[j]], add=True)

    gather(0, b0, g0)

    def body(jo, carry):
        j = 2 * jo
        gather(j + 1, b1, g1)
        wait_gather(b0, g0)
        scatter(j, b0)
        gather(j + 2, b0, g0)
        wait_gather(b1, g1)
        scatter(j + 1, b1)
        return carry

    lax.fori_loop(0, (_NCHUNK - 1) // 2, body, 0)

    wait_gather(b0, g0)
    scatter(_NCHUNK - 1, b0)

    plsc.subcore_barrier()
    pltpu.sync_copy(acc_sh.at[pl.ds(row0, _ROWS_PER_TILE)],
                    out_hbm.at[cid, pl.ds(row0, _ROWS_PER_TILE)])


_SC_SCATTER_CACHE = []


def _get_sc_scatter():
    if not _SC_SCATTER_CACHE:
        _SC_SCATTER_CACHE.append(pl.kernel(
            _sc_scatter_body,
            out_type=jax.ShapeDtypeStruct((_NC, _N_PAD, IN_DIM),
                                          jnp.float32),
            mesh=plsc.VectorSubcoreMesh(core_axis_name="c",
                                        subcore_axis_name="s"),
            scratch_types=[
                pltpu.VMEM((_PER_W,), jnp.int32),
                pltpu.VMEM((_NCHUNK, _CHUNK), jnp.int32),
                pltpu.VMEM((_CHUNK, IN_DIM), jnp.float32),
                pltpu.VMEM((_CHUNK, IN_DIM), jnp.float32),
                pltpu.VMEM_SHARED((_N_PAD, IN_DIM), jnp.float32),
                pltpu.SemaphoreType.DMA,
                pltpu.SemaphoreType.DMA,
            ],
        ))
    return _SC_SCATTER_CACHE[0]


def kernel(x, edge_index, batch, W_rel1, b_rel1, W_root1,
           W_rel2, b_rel2, W_root2, W_lin, b_lin):
    src = edge_index[0].astype(jnp.int32).reshape(_NW, _PER_W)
    dst = edge_index[1].astype(jnp.int32).reshape(_NW, _NCHUNK, _CHUNK)
    batch2d = batch.astype(jnp.int32).reshape(1, N_NODES)
    zeros_pad = jnp.zeros((_N_PAD, IN_DIM), jnp.float32)
    W_rel2p = jnp.pad(W_rel2, ((0, 0), (0, IN_DIM - HID)))

    sc_scatter = _get_sc_scatter()
    xo1 = _pre(x, W_root1, b_rel1.reshape(1, HID))
    p1 = sc_scatter(x, src, dst, zeros_pad)
    xr2p, xo2 = _mid(p1, xo1, W_rel1,
                     W_rel2p, W_root2, b_rel2.reshape(1, HID))
    p2 = sc_scatter(xr2p, src, dst, zeros_pad)
    return _pool(p2, xo2, batch2d, W_lin, b_lin.reshape(1, NUM_CLASSES))


# speedup vs baseline: 1.6309x; 1.0002x over previous
import jax
import jax.numpy as jnp
from jax import lax
from jax.experimental import pallas as pl
from jax.experimental.pallas import tpu as pltpu
from jax.experimental.pallas import tpu_sc as plsc

N_NODES = 10000
N_EDGES = 320000
IN_DIM = 128
HID = 64
NUM_CLASSES = 2
NUM_GRAPHS = 512

_NC = 2
_NS = 16
_NW = _NC * _NS
_PER_W = N_EDGES // _NW
_CHUNK = 80
_NCHUNK = _PER_W // _CHUNK
_N_PAD = 10240
_ROWS_PER_TILE = _N_PAD // _NS


def _pre_body(x_ref, wo1_ref, b1_ref, xo1_ref):
    xo1_ref[...] = (jnp.dot(x_ref[...], wo1_ref[...],
                            preferred_element_type=jnp.float32) + b1_ref[...])


def _mid_body(p_ref, xo1_ref, wr1_ref, wo2_ref,
              b2_ref, h1p_ref, xo2_ref):
    agg1 = p_ref[0, :N_NODES] + p_ref[1, :N_NODES]
    h1 = jnp.maximum(
        jnp.dot(agg1, wr1_ref[...], preferred_element_type=jnp.float32)
        + xo1_ref[...],
        0.0)
    h1p_ref[...] = jnp.concatenate(
        [h1, jnp.zeros((N_NODES, IN_DIM - HID), jnp.float32)], axis=1)
    xo2_ref[...] = (jnp.dot(h1, wo2_ref[...],
                            preferred_element_type=jnp.float32) + b2_ref[...])


def _pool_body(p_ref, xo2_ref, wr2_ref, batch_ref, wl_ref, bl_ref, out_ref):
    agg2 = p_ref[0, :N_NODES, :HID] + p_ref[1, :N_NODES, :HID]
    h2 = jnp.maximum(
        jnp.dot(agg2, wr2_ref[...], preferred_element_type=jnp.float32)
        + xo2_ref[...], 0.0)
    gid = lax.broadcasted_iota(jnp.int32, (NUM_GRAPHS, 1), 0)
    sums = jnp.zeros((NUM_GRAPHS, HID), jnp.float32)
    counts = jnp.zeros((NUM_GRAPHS, 1), jnp.float32)
    ch = 1000
    for k in range(N_NODES // ch):
        b = batch_ref[:, k * ch:(k + 1) * ch]
        m = (gid == b).astype(jnp.float32)
        sums = sums + jnp.dot(m, h2[k * ch:(k + 1) * ch, :],
                              preferred_element_type=jnp.float32,
                              precision=lax.Precision.HIGHEST)
        counts = counts + jnp.sum(m, axis=1, keepdims=True)
    pooled = sums / jnp.maximum(counts, 1.0)
    out_ref[...] = (jnp.dot(pooled, wl_ref[...],
                            preferred_element_type=jnp.float32,
                            precision=lax.Precision.HIGHEST) + bl_ref[...])


_pre = pl.pallas_call(
    _pre_body,
    out_shape=jax.ShapeDtypeStruct((N_NODES, HID), jnp.float32),
)

_mid = pl.pallas_call(
    _mid_body,
    out_shape=[jax.ShapeDtypeStruct((N_NODES, IN_DIM), jnp.float32),
               jax.ShapeDtypeStruct((N_NODES, HID), jnp.float32)],
)

_pool = pl.pallas_call(
    _pool_body,
    out_shape=jax.ShapeDtypeStruct((NUM_GRAPHS, NUM_CLASSES), jnp.float32),
)


def _sc_scatter_body(xr_hbm, src_hbm, dst_hbm, zeros_hbm, out_hbm,
                     src_all, dst_all, b0, b1, acc_sh, g0, g1):
    cid = lax.axis_index("c")
    sid = lax.axis_index("s")
    wid = cid * _NS + sid
    row0 = sid * _ROWS_PER_TILE

    pltpu.sync_copy(src_hbm.at[wid], src_all)
    pltpu.sync_copy(dst_hbm.at[wid], dst_all)
    pltpu.sync_copy(zeros_hbm.at[pl.ds(row0, _ROWS_PER_TILE)],
                    acc_sh.at[pl.ds(row0, _ROWS_PER_TILE)])
    plsc.subcore_barrier()

    def gather(j, buf, sem):
        pltpu.async_copy(
            xr_hbm.at[src_all.at[pl.ds(pl.multiple_of(j * _CHUNK, 8),
                                       _CHUNK)]], buf, sem)

    def wait_gather(buf, sem):
        pltpu.make_async_copy(xr_hbm.at[src_all.at[pl.ds(0, _CHUNK)]],
                              buf, sem).wait()

    def scatter(j, buf):
        pltpu.sync_copy(buf, acc_sh.at[dst_all.at[j]], add=True)

    gather(0, b0, g0)

    def body(jo, carry):
        j = 2 * jo
        gather(j + 1, b1, g1)
        wait_gather(b0, g0)
        scatter(j, b0)
        gather(j + 2, b0, g0)
        wait_gather(b1, g1)
        scatter(j + 1, b1)
        return carry

    lax.fori_loop(0, (_NCHUNK - 1) // 2, body, 0)

    wait_gather(b0, g0)
    scatter(_NCHUNK - 1, b0)

    plsc.subcore_barrier()
    pltpu.sync_copy(acc_sh.at[pl.ds(row0, _ROWS_PER_TILE)],
                    out_hbm.at[cid, pl.ds(row0, _ROWS_PER_TILE)])


_SC_SCATTER_CACHE = []


def _get_sc_scatter():
    if not _SC_SCATTER_CACHE:
        _SC_SCATTER_CACHE.append(pl.kernel(
            _sc_scatter_body,
            out_type=jax.ShapeDtypeStruct((_NC, _N_PAD, IN_DIM),
                                          jnp.float32),
            mesh=plsc.VectorSubcoreMesh(core_axis_name="c",
                                        subcore_axis_name="s"),
            scratch_types=[
                pltpu.VMEM((_PER_W,), jnp.int32),
                pltpu.VMEM((_NCHUNK, _CHUNK), jnp.int32),
                pltpu.VMEM((_CHUNK, IN_DIM), jnp.float32),
                pltpu.VMEM((_CHUNK, IN_DIM), jnp.float32),
                pltpu.VMEM_SHARED((_N_PAD, IN_DIM), jnp.float32),
                pltpu.SemaphoreType.DMA,
                pltpu.SemaphoreType.DMA,
            ],
        ))
    return _SC_SCATTER_CACHE[0]


def kernel(x, edge_index, batch, W_rel1, b_rel1, W_root1,
           W_rel2, b_rel2, W_root2, W_lin, b_lin):
    src = edge_index[0].astype(jnp.int32).reshape(_NW, _PER_W)
    dst = edge_index[1].astype(jnp.int32).reshape(_NW, _NCHUNK, _CHUNK)
    batch2d = batch.astype(jnp.int32).reshape(1, N_NODES)
    zeros_pad = jnp.zeros((_N_PAD, IN_DIM), jnp.float32)

    sc_scatter = _get_sc_scatter()
    xo1 = _pre(x, W_root1, b_rel1.reshape(1, HID))
    p1 = sc_scatter(x, src, dst, zeros_pad)
    h1p, xo2 = _mid(p1, xo1, W_rel1, W_root2, b_rel2.reshape(1, HID))
    p2 = sc_scatter(h1p, src, dst, zeros_pad)
    return _pool(p2, xo2, W_rel2, batch2d, W_lin,
                 b_lin.reshape(1, NUM_CLASSES))
